# Initial kernel scaffold; baseline (speedup 1.0000x reference)
import jax
import jax.numpy as jnp
from jax.experimental import pallas as pl


def kernel(xin, codebooks):
    raise NotImplementedError("write your pallas kernel here")


# speedup vs baseline: 1.7364x; 1.7364x over previous
import functools

import jax
import jax.numpy as jnp
from jax.experimental import pallas as pl

N_CODES = 1024
N_GROUPS = 2
DIM = 512
E_DIM = DIM // N_GROUPS
LAYERS = 2
B = 16
T = 1024
TB = 512

_PREC = jax.lax.Precision.HIGHEST


def _vq_body(x_ref, cb_ref, cbt_ref,
             qout_ref, i00_ref, i01_ref, i10_ref, i11_ref, ss0_ref, ss1_ref):
    b = pl.program_id(0)
    t = pl.program_id(1)

    @pl.when((b == 0) & (t == 0))
    def _init():
        ss0_ref[...] = jnp.zeros((1, 1), jnp.float32)
        ss1_ref[...] = jnp.zeros((1, 1), jnp.float32)

    idx_refs = ((i00_ref, i01_ref), (i10_ref, i11_ref))
    residual = x_ref[0]
    qout = None
    for layer in range(LAYERS):
        zq_parts = []
        for g in range(N_GROUPS):
            w = cb_ref[layer, g]
            wt = cbt_ref[layer, g]
            xg = residual[g * E_DIM:(g + 1) * E_DIM, :]
            xn = jnp.sum(xg * xg, axis=0, keepdims=True)
            wn = jnp.sum(w * w, axis=1, keepdims=True)
            scores = jax.lax.dot_general(
                w, xg, (((1,), (0,)), ((), ())))
            d = (xn + wn) - 2.0 * scores
            dmin = jnp.min(d, axis=0, keepdims=True)
            iota = jax.lax.broadcasted_iota(jnp.int32, d.shape, 0)
            idx = jnp.min(jnp.where(d == dmin, iota, jnp.int32(N_CODES)),
                          axis=0, keepdims=True)
            idx_refs[layer][g][0] = idx
            onehot = (iota == idx).astype(jnp.float32)
            zq_parts.append(jax.lax.dot_general(
                wt, onehot, (((1,), (0,)), ((), ())), precision=_PREC))
        zq = jnp.concatenate(zq_parts, axis=0)
        delta = zq - residual
        ss = jnp.sum(delta * delta).reshape(1, 1)
        if layer == 0:
            ss0_ref[...] += ss
        else:
            ss1_ref[...] += ss
        q = residual + delta
        residual = residual - q
        qout = q if qout is None else qout + q
    qout_ref[0] = qout


@jax.jit
def kernel(xin, codebooks):
    cbt = jnp.swapaxes(codebooks, 2, 3)
    grid = (B, T // TB)
    outs = pl.pallas_call(
        _vq_body,
        grid=grid,
        in_specs=[
            pl.BlockSpec((1, DIM, TB), lambda b, t: (b, 0, t)),
            pl.BlockSpec((LAYERS, N_GROUPS, N_CODES, E_DIM),
                         lambda b, t: (0, 0, 0, 0)),
            pl.BlockSpec((LAYERS, N_GROUPS, E_DIM, N_CODES),
                         lambda b, t: (0, 0, 0, 0)),
        ],
        out_specs=[
            pl.BlockSpec((1, DIM, TB), lambda b, t: (b, 0, t)),
            pl.BlockSpec((1, 1, TB), lambda b, t: (b, 0, t)),
            pl.BlockSpec((1, 1, TB), lambda b, t: (b, 0, t)),
            pl.BlockSpec((1, 1, TB), lambda b, t: (b, 0, t)),
            pl.BlockSpec((1, 1, TB), lambda b, t: (b, 0, t)),
            pl.BlockSpec((1, 1), lambda b, t: (0, 0)),
            pl.BlockSpec((1, 1), lambda b, t: (0, 0)),
        ],
        out_shape=[
            jax.ShapeDtypeStruct((B, DIM, T), jnp.float32),
            jax.ShapeDtypeStruct((B, 1, T), jnp.int32),
            jax.ShapeDtypeStruct((B, 1, T), jnp.int32),
            jax.ShapeDtypeStruct((B, 1, T), jnp.int32),
            jax.ShapeDtypeStruct((B, 1, T), jnp.int32),
            jax.ShapeDtypeStruct((1, 1), jnp.float32),
            jax.ShapeDtypeStruct((1, 1), jnp.float32),
        ],
    )(xin, codebooks, cbt)
    qout, i00, i01, i10, i11, ss0, ss1 = outs
    n_elem = B * T * DIM
    loss = ((ss0[0, 0] + ss1[0, 0])
            * jnp.float32((1.0 + 0.25) / (LAYERS * n_elem)))
    indices = jnp.stack([i.reshape(B * T) for i in (i00, i01, i10, i11)])
    return qout, loss, indices


# speedup vs baseline: 2.5238x; 1.4534x over previous
import jax
import jax.numpy as jnp
from jax.experimental import pallas as pl

N_CODES = 1024
N_GROUPS = 2
DIM = 512
E_DIM = DIM // N_GROUPS
LAYERS = 2
B = 16
T = 1024
TB = 512


def _vq_body(x_ref, cb_ref, cbt0_ref, cbt1_ref, cbt2_ref,
             qout_ref, i00_ref, i01_ref, i10_ref, i11_ref, ss0_ref, ss1_ref):
    b = pl.program_id(0)
    t = pl.program_id(1)

    @pl.when((b == 0) & (t == 0))
    def _init():
        ss0_ref[...] = jnp.zeros((1, 1), jnp.float32)
        ss1_ref[...] = jnp.zeros((1, 1), jnp.float32)

    idx_refs = ((i00_ref, i01_ref), (i10_ref, i11_ref))
    residual = x_ref[0]
    qout = None
    for layer in range(LAYERS):
        zq_parts = []
        for g in range(N_GROUPS):
            w = cb_ref[layer, g]
            xg = residual[g * E_DIM:(g + 1) * E_DIM, :]
            xn = jnp.sum(xg * xg, axis=0, keepdims=True)
            wn = jnp.sum(w * w, axis=1, keepdims=True)
            scores = jax.lax.dot_general(
                w, xg, (((1,), (0,)), ((), ())))
            d = (xn + wn) - 2.0 * scores
            dmin = jnp.min(d, axis=0, keepdims=True)
            iota = jax.lax.broadcasted_iota(jnp.int32, d.shape, 0)
            idx = jnp.min(jnp.where(d == dmin, iota, jnp.int32(N_CODES)),
                          axis=0, keepdims=True)
            idx_refs[layer][g][0] = idx
            onehot = (iota == idx).astype(jnp.bfloat16)
            zq_g = None
            for cbt_ref in (cbt0_ref, cbt1_ref, cbt2_ref):
                part = jax.lax.dot_general(
                    cbt_ref[layer, g], onehot, (((1,), (0,)), ((), ())),
                    preferred_element_type=jnp.float32)
                zq_g = part if zq_g is None else zq_g + part
            zq_parts.append(zq_g)
        zq = jnp.concatenate(zq_parts, axis=0)
        delta = zq - residual
        ss = jnp.sum(delta * delta).reshape(1, 1)
        if layer == 0:
            ss0_ref[...] += ss
        else:
            ss1_ref[...] += ss
        q = residual + delta
        residual = residual - q
        qout = q if qout is None else qout + q
    qout_ref[0] = qout


@jax.jit
def kernel(xin, codebooks):
    cbt = jnp.swapaxes(codebooks, 2, 3)
    cbt0 = cbt.astype(jnp.bfloat16)
    r1 = cbt - cbt0.astype(jnp.float32)
    cbt1 = r1.astype(jnp.bfloat16)
    cbt2 = (r1 - cbt1.astype(jnp.float32)).astype(jnp.bfloat16)
    grid = (B, T // TB)
    cbt_spec = pl.BlockSpec((LAYERS, N_GROUPS, E_DIM, N_CODES),
                            lambda b, t: (0, 0, 0, 0))
    outs = pl.pallas_call(
        _vq_body,
        grid=grid,
        in_specs=[
            pl.BlockSpec((1, DIM, TB), lambda b, t: (b, 0, t)),
            pl.BlockSpec((LAYERS, N_GROUPS, N_CODES, E_DIM),
                         lambda b, t: (0, 0, 0, 0)),
            cbt_spec, cbt_spec, cbt_spec,
        ],
        out_specs=[
            pl.BlockSpec((1, DIM, TB), lambda b, t: (b, 0, t)),
            pl.BlockSpec((1, 1, TB), lambda b, t: (b, 0, t)),
            pl.BlockSpec((1, 1, TB), lambda b, t: (b, 0, t)),
            pl.BlockSpec((1, 1, TB), lambda b, t: (b, 0, t)),
            pl.BlockSpec((1, 1, TB), lambda b, t: (b, 0, t)),
            pl.BlockSpec((1, 1), lambda b, t: (0, 0)),
            pl.BlockSpec((1, 1), lambda b, t: (0, 0)),
        ],
        out_shape=[
            jax.ShapeDtypeStruct((B, DIM, T), jnp.float32),
            jax.ShapeDtypeStruct((B, 1, T), jnp.int32),
            jax.ShapeDtypeStruct((B, 1, T), jnp.int32),
            jax.ShapeDtypeStruct((B, 1, T), jnp.int32),
            jax.ShapeDtypeStruct((B, 1, T), jnp.int32),
            jax.ShapeDtypeStruct((1, 1), jnp.float32),
            jax.ShapeDtypeStruct((1, 1), jnp.float32),
        ],
    )(xin, codebooks, cbt0, cbt1, cbt2)
    qout, i00, i01, i10, i11, ss0, ss1 = outs
    n_elem = B * T * DIM
    loss = ((ss0[0, 0] + ss1[0, 0])
            * jnp.float32((1.0 + 0.25) / (LAYERS * n_elem)))
    indices = jnp.stack([i.reshape(B * T) for i in (i00, i01, i10, i11)])
    return qout, loss, indices


# speedup vs baseline: 2.6223x; 1.0391x over previous
import jax
import jax.numpy as jnp
from jax.experimental import pallas as pl
from jax.experimental.pallas import tpu as pltpu

N_CODES = 1024
N_GROUPS = 2
DIM = 512
E_DIM = DIM // N_GROUPS
LAYERS = 2
B = 16
T = 1024
TB = 512


def _vq_body(x_ref, cb_ref, cbt_ref,
             qout_ref, i00_ref, i01_ref, i10_ref, i11_ref, ss0_ref, ss1_ref,
             cbt0_ref, cbt1_ref, cbt2_ref):
    b = pl.program_id(0)
    t = pl.program_id(1)

    @pl.when((b == 0) & (t == 0))
    def _init():
        ss0_ref[...] = jnp.zeros((1, 1), jnp.float32)
        ss1_ref[...] = jnp.zeros((1, 1), jnp.float32)
        w = cbt_ref[...]
        w0 = w.astype(jnp.bfloat16)
        r1 = w - w0.astype(jnp.float32)
        w1 = r1.astype(jnp.bfloat16)
        w2 = (r1 - w1.astype(jnp.float32)).astype(jnp.bfloat16)
        cbt0_ref[...] = w0
        cbt1_ref[...] = w1
        cbt2_ref[...] = w2

    idx_refs = ((i00_ref, i01_ref), (i10_ref, i11_ref))
    residual = x_ref[0]
    qout = None
    for layer in range(LAYERS):
        zq_parts = []
        for g in range(N_GROUPS):
            w = cb_ref[layer, g]
            xg = residual[g * E_DIM:(g + 1) * E_DIM, :]
            xn = jnp.sum(xg * xg, axis=0, keepdims=True)
            wn = jnp.sum(w * w, axis=1, keepdims=True)
            scores = jax.lax.dot_general(
                w, xg, (((1,), (0,)), ((), ())))
            d = (xn + wn) - 2.0 * scores
            dmin = jnp.min(d, axis=0, keepdims=True)
            iota = jax.lax.broadcasted_iota(jnp.int32, d.shape, 0)
            idx = jnp.min(jnp.where(d == dmin, iota, jnp.int32(N_CODES)),
                          axis=0, keepdims=True)
            idx_refs[layer][g][0] = idx
            onehot = (iota == idx).astype(jnp.bfloat16)
            zq_g = None
            for part_ref in (cbt0_ref, cbt1_ref, cbt2_ref):
                part = jax.lax.dot_general(
                    part_ref[layer, g], onehot, (((1,), (0,)), ((), ())),
                    preferred_element_type=jnp.float32)
                zq_g = part if zq_g is None else zq_g + part
            zq_parts.append(zq_g)
        zq = jnp.concatenate(zq_parts, axis=0)
        delta = zq - residual
        ss = jnp.sum(delta * delta).reshape(1, 1)
        if layer == 0:
            ss0_ref[...] += ss
        else:
            ss1_ref[...] += ss
        q = residual + delta
        residual = residual - q
        qout = q if qout is None else qout + q
    qout_ref[0] = qout


@jax.jit
def kernel(xin, codebooks):
    cbt = jnp.swapaxes(codebooks, 2, 3)
    grid = (B, T // TB)
    cbt_sh = (LAYERS, N_GROUPS, E_DIM, N_CODES)
    outs = pl.pallas_call(
        _vq_body,
        grid=grid,
        in_specs=[
            pl.BlockSpec((1, DIM, TB), lambda b, t: (b, 0, t)),
            pl.BlockSpec((LAYERS, N_GROUPS, N_CODES, E_DIM),
                         lambda b, t: (0, 0, 0, 0)),
            pl.BlockSpec(cbt_sh, lambda b, t: (0, 0, 0, 0)),
        ],
        scratch_shapes=[pltpu.VMEM(cbt_sh, jnp.bfloat16)] * 3,
        out_specs=[
            pl.BlockSpec((1, DIM, TB), lambda b, t: (b, 0, t)),
            pl.BlockSpec((1, 1, TB), lambda b, t: (b, 0, t)),
            pl.BlockSpec((1, 1, TB), lambda b, t: (b, 0, t)),
            pl.BlockSpec((1, 1, TB), lambda b, t: (b, 0, t)),
            pl.BlockSpec((1, 1, TB), lambda b, t: (b, 0, t)),
            pl.BlockSpec((1, 1), lambda b, t: (0, 0)),
            pl.BlockSpec((1, 1), lambda b, t: (0, 0)),
        ],
        out_shape=[
            jax.ShapeDtypeStruct((B, DIM, T), jnp.float32),
            jax.ShapeDtypeStruct((B, 1, T), jnp.int32),
            jax.ShapeDtypeStruct((B, 1, T), jnp.int32),
            jax.ShapeDtypeStruct((B, 1, T), jnp.int32),
            jax.ShapeDtypeStruct((B, 1, T), jnp.int32),
            jax.ShapeDtypeStruct((1, 1), jnp.float32),
            jax.ShapeDtypeStruct((1, 1), jnp.float32),
        ],
    )(xin, codebooks, cbt)
    qout, i00, i01, i10, i11, ss0, ss1 = outs
    n_elem = B * T * DIM
    loss = ((ss0[0, 0] + ss1[0, 0])
            * jnp.float32((1.0 + 0.25) / (LAYERS * n_elem)))
    indices = jnp.stack([i.reshape(B * T) for i in (i00, i01, i10, i11)])
    return qout, loss, indices


# speedup vs baseline: 2.8205x; 1.0756x over previous
import jax
import jax.numpy as jnp
from jax.experimental import pallas as pl
from jax.experimental.pallas import tpu as pltpu

N_CODES = 1024
N_GROUPS = 2
DIM = 512
E_DIM = DIM // N_GROUPS
LAYERS = 2
B = 16
T = 1024
TB = 1024


def _vq_body(x_ref, cb_ref, cbt_ref,
             qout_ref, i00_ref, i01_ref, i10_ref, i11_ref, ss0_ref, ss1_ref,
             cbt0_ref, cbt1_ref, cbt2_ref):
    b = pl.program_id(0)
    t = pl.program_id(1)

    @pl.when((b == 0) & (t == 0))
    def _init():
        ss0_ref[...] = jnp.zeros((1, 1), jnp.float32)
        ss1_ref[...] = jnp.zeros((1, 1), jnp.float32)
        w = cbt_ref[...]
        w0 = w.astype(jnp.bfloat16)
        r1 = w - w0.astype(jnp.float32)
        w1 = r1.astype(jnp.bfloat16)
        w2 = (r1 - w1.astype(jnp.float32)).astype(jnp.bfloat16)
        cbt0_ref[...] = w0
        cbt1_ref[...] = w1
        cbt2_ref[...] = w2

    idx_refs = ((i00_ref, i01_ref), (i10_ref, i11_ref))
    residual = x_ref[0]
    qout = None
    for layer in range(LAYERS):
        zq_parts = []
        for g in range(N_GROUPS):
            w = cb_ref[layer, g]
            xg = residual[g * E_DIM:(g + 1) * E_DIM, :]
            xn = jnp.sum(xg * xg, axis=0, keepdims=True)
            wn = jnp.sum(w * w, axis=1, keepdims=True)
            scores = jax.lax.dot_general(
                w, xg, (((1,), (0,)), ((), ())))
            d = (xn + wn) - 2.0 * scores
            dmin = jnp.min(d, axis=0, keepdims=True)
            iota = jax.lax.broadcasted_iota(jnp.int32, d.shape, 0)
            idx = jnp.min(jnp.where(d == dmin, iota, jnp.int32(N_CODES)),
                          axis=0, keepdims=True)
            idx_refs[layer][g][0] = idx
            onehot = (iota == idx).astype(jnp.bfloat16)
            zq_g = None
            for part_ref in (cbt0_ref, cbt1_ref, cbt2_ref):
                part = jax.lax.dot_general(
                    part_ref[layer, g], onehot, (((1,), (0,)), ((), ())),
                    preferred_element_type=jnp.float32)
                zq_g = part if zq_g is None else zq_g + part
            zq_parts.append(zq_g)
        zq = jnp.concatenate(zq_parts, axis=0)
        delta = zq - residual
        ss = jnp.sum(delta * delta).reshape(1, 1)
        if layer == 0:
            ss0_ref[...] += ss
        else:
            ss1_ref[...] += ss
        q = residual + delta
        residual = residual - q
        qout = q if qout is None else qout + q
    qout_ref[0] = qout


@jax.jit
def kernel(xin, codebooks):
    cbt = jnp.swapaxes(codebooks, 2, 3)
    grid = (B, T // TB)
    cbt_sh = (LAYERS, N_GROUPS, E_DIM, N_CODES)
    outs = pl.pallas_call(
        _vq_body,
        grid=grid,
        in_specs=[
            pl.BlockSpec((1, DIM, TB), lambda b, t: (b, 0, t)),
            pl.BlockSpec((LAYERS, N_GROUPS, N_CODES, E_DIM),
                         lambda b, t: (0, 0, 0, 0)),
            pl.BlockSpec(cbt_sh, lambda b, t: (0, 0, 0, 0)),
        ],
        scratch_shapes=[pltpu.VMEM(cbt_sh, jnp.bfloat16)] * 3,
        out_specs=[
            pl.BlockSpec((1, DIM, TB), lambda b, t: (b, 0, t)),
            pl.BlockSpec((1, 1, TB), lambda b, t: (b, 0, t)),
            pl.BlockSpec((1, 1, TB), lambda b, t: (b, 0, t)),
            pl.BlockSpec((1, 1, TB), lambda b, t: (b, 0, t)),
            pl.BlockSpec((1, 1, TB), lambda b, t: (b, 0, t)),
            pl.BlockSpec((1, 1), lambda b, t: (0, 0)),
            pl.BlockSpec((1, 1), lambda b, t: (0, 0)),
        ],
        out_shape=[
            jax.ShapeDtypeStruct((B, DIM, T), jnp.float32),
            jax.ShapeDtypeStruct((B, 1, T), jnp.int32),
            jax.ShapeDtypeStruct((B, 1, T), jnp.int32),
            jax.ShapeDtypeStruct((B, 1, T), jnp.int32),
            jax.ShapeDtypeStruct((B, 1, T), jnp.int32),
            jax.ShapeDtypeStruct((1, 1), jnp.float32),
            jax.ShapeDtypeStruct((1, 1), jnp.float32),
        ],
    )(xin, codebooks, cbt)
    qout, i00, i01, i10, i11, ss0, ss1 = outs
    n_elem = B * T * DIM
    loss = ((ss0[0, 0] + ss1[0, 0])
            * jnp.float32((1.0 + 0.25) / (LAYERS * n_elem)))
    indices = jnp.stack([i.reshape(B * T) for i in (i00, i01, i10, i11)])
    return qout, loss, indices
